# baseline (speedup 1.0000x reference)
import functools

import jax
import jax.numpy as jnp
from jax import lax
from jax.experimental import pallas as pl
from jax.experimental.pallas import tpu as pltpu
from jax.experimental.pallas import tpu_sc as plsc

N = 10000
E = 320000
D_MODEL = 128
H = 8
DH = 16
D_EDGE = 16

NC = 2
NS = 16
NW = NC * NS
B = 40
CHUNKS = E // (NW * B)
EPW = E // NW
NP = 10240
RPS = NP // NS


def _ds16(h):
    return pl.ds(h * 16, 16)


def _proj_body(x_ref, w_ref, b_ref, q_ref, k_ref, v_ref, xr_ref):
    p = jnp.dot(x_ref[...], w_ref[...], preferred_element_type=jnp.float32)
    p = p + b_ref[...]
    q_ref[...] = p[:, 0:128]
    k_ref[...] = p[:, 128:256]
    v_ref[...] = p[:, 256:384]
    xr_ref[...] = p[:, 384:512]


def _proj(x, Wcat, bcat):
    BN = 1000
    outs = [jax.ShapeDtypeStruct((N, 128), jnp.float32)] * 4
    return pl.pallas_call(
        _proj_body,
        grid=(N // BN,),
        in_specs=[
            pl.BlockSpec((BN, 128), lambda i: (i, 0)),
            pl.BlockSpec((128, 512), lambda i: (0, 0)),
            pl.BlockSpec((1, 512), lambda i: (0, 0)),
        ],
        out_specs=[pl.BlockSpec((BN, 128), lambda i: (i, 0))] * 4,
        out_shape=outs,
    )(x, Wcat, bcat)


def _eproj_body(a_ref, w_ref, o_ref):
    o_ref[...] = jnp.dot(a_ref[...], w_ref[...],
                         preferred_element_type=jnp.float32)


def _eproj(edge_attr, We):
    BE = 3200
    return pl.pallas_call(
        _eproj_body,
        grid=(E // BE,),
        in_specs=[
            pl.BlockSpec((BE, 16), lambda i: (i, 0)),
            pl.BlockSpec((16, 128), lambda i: (0, 0)),
        ],
        out_specs=pl.BlockSpec((BE, 128), lambda i: (i, 0)),
        out_shape=jax.ShapeDtypeStruct((E, 128), jnp.float32),
    )(edge_attr, We)


def _sc_body(q4_hbm, k_hbm, v_hbm, e_hbm, idx_hbm, zer128_hbm,
             den_out, outp_hbm,
             den_sh, acc_sh, buf1, buf2, buf3, vbuf, exbuf, denbuf, idxb,
             sem1, sem2, semA, semD):
    c = lax.axis_index("c")
    s = lax.axis_index("s")
    wid = c * NS + s
    rows = pl.ds(s * RPS, RPS)
    drows = pl.ds(s * (NP // 8 // NS), NP // 8 // NS)

    pltpu.sync_copy(zer128_hbm.at[drows], den_sh.at[drows])
    pltpu.sync_copy(zer128_hbm.at[rows], acc_sh.at[rows])

    lane = lax.iota(jnp.int32, 16)
    zv = jnp.zeros((16,), jnp.float32)

    plsc.subcore_barrier()

    def _chunk(t, _):
        base = wid * EPW + t * B

        @pl.when(t > 0)
        def _():
            pltpu.make_async_copy(buf1, acc_sh.at[idxb.at[1]], semA).wait()
            pltpu.make_async_copy(denbuf, den_sh.at[idxb.at[2]], semD).wait()

        pltpu.sync_copy(idx_hbm.at[wid, t], idxb)
        cq = pltpu.async_copy(q4_hbm.at[idxb.at[1]], buf1, sem1)
        ck = pltpu.async_copy(k_hbm.at[idxb.at[0]], buf3, sem1)
        ce = pltpu.async_copy(e_hbm.at[pl.ds(base, B)], buf2, sem1)
        cv = pltpu.async_copy(v_hbm.at[idxb.at[0]], vbuf, sem2)
        cq.wait()
        ck.wait()
        ce.wait()

        def _edge(i, _):
            av = jnp.zeros((16,), jnp.float32)
            for h in range(H):
                qv = buf1[i, _ds16(h)]
                w = buf2[i, _ds16(h)] + buf3[i, _ds16(h)]
                p = qv * w
                for sh in (8, 4, 2, 1):
                    p = p + p.at[lane ^ sh].get(mode="promise_in_bounds")
                av = jnp.where(lane == h, p, av)
            exbuf[i, :] = jnp.exp(av)
            return _
        lax.fori_loop(0, B, _edge, None)

        for j in (0, 16, B - 16):
            ov = idxb[3, pl.ds(j, 16)]
            for r in range(16):
                i = j + r
                ev = exbuf[i, :]
                for blk in range(8):
                    denbuf[i, _ds16(blk)] = zv
                denbuf[i, pl.ds(ov[r], 16)] = ev

        pltpu.async_copy(denbuf, den_sh.at[idxb.at[2]], semD, add=True)
        cv.wait()

        def _scale(i, _):
            tv = exbuf[i, :]
            for h in range(H):
                buf1[i, _ds16(h)] = tv[h] * (vbuf[i, _ds16(h)]
                                             + buf2[i, _ds16(h)])
            return _
        lax.fori_loop(0, B, _scale, None)

        pltpu.async_copy(buf1, acc_sh.at[idxb.at[1]], semA, add=True)
        return _
    lax.fori_loop(0, CHUNKS, _chunk, None)

    pltpu.make_async_copy(buf1, acc_sh.at[idxb.at[1]], semA).wait()
    pltpu.make_async_copy(denbuf, den_sh.at[idxb.at[2]], semD).wait()

    plsc.subcore_barrier()
    pltpu.sync_copy(den_sh.at[drows], den_out.at[c, drows])
    pltpu.sync_copy(acc_sh.at[rows], outp_hbm.at[c, rows])


def _sc_edges(q4, ktab, vtab, etab, idx_pack, zer128):
    mesh = plsc.VectorSubcoreMesh(core_axis_name="c", subcore_axis_name="s")
    f = pl.kernel(
        _sc_body,
        out_type=(jax.ShapeDtypeStruct((NC, NP // 8, 128), jnp.float32),
                  jax.ShapeDtypeStruct((NC, NP, 128), jnp.float32)),
        mesh=mesh,
        compiler_params=pltpu.CompilerParams(needs_layout_passes=False),
        scratch_types=[
            pltpu.VMEM_SHARED((NP // 8, 128), jnp.float32),
            pltpu.VMEM_SHARED((NP, 128), jnp.float32),
            pltpu.VMEM((B, 128), jnp.float32),
            pltpu.VMEM((B, 128), jnp.float32),
            pltpu.VMEM((B, 128), jnp.float32),
            pltpu.VMEM((B, 128), jnp.float32),
            pltpu.VMEM((B, 16), jnp.float32),
            pltpu.VMEM((B, 128), jnp.float32),
            pltpu.VMEM((4, B), jnp.int32),
            pltpu.SemaphoreType.DMA,
            pltpu.SemaphoreType.DMA,
            pltpu.SemaphoreType.DMA,
            pltpu.SemaphoreType.DMA,
        ],
    )
    return f(q4, ktab, vtab, etab, idx_pack, zer128)


def _fin1_body(outp_ref, den_ref, xr_ref, wbo_ref, wbx_ref, out2_ref,
               psum_ref):
    i = pl.program_id(0)
    dsum = den_ref[0] + den_ref[1] + 1e-16
    dexp = jnp.repeat(dsum[:, 0:8], 16, axis=1)
    o = (outp_ref[0] + outp_ref[1]) / dexp
    xr = xr_ref[...]
    t = jnp.sum(o * wbo_ref[...] + xr * wbx_ref[...], axis=1, keepdims=True)
    g = 1.0 / (1.0 + jnp.exp(-t))
    out2 = g * xr + (1.0 - g) * o
    out2_ref[...] = out2

    @pl.when(i == 0)
    def _():
        psum_ref[...] = jnp.zeros((8, 128), jnp.float32)

    psum_ref[0:1, :] += jnp.sum(out2, axis=0, keepdims=True)
    psum_ref[1:2, :] += jnp.sum(out2 * out2, axis=0, keepdims=True)


def _fin1(outp, den, xr, wbo, wbx):
    BN = 2000
    return pl.pallas_call(
        _fin1_body,
        grid=(N // BN,),
        in_specs=[
            pl.BlockSpec((2, BN, 128), lambda i: (0, i, 0)),
            pl.BlockSpec((2, BN, 16), lambda i: (0, i, 0)),
            pl.BlockSpec((BN, 128), lambda i: (i, 0)),
            pl.BlockSpec((1, 128), lambda i: (0, 0)),
            pl.BlockSpec((1, 128), lambda i: (0, 0)),
        ],
        out_specs=[
            pl.BlockSpec((BN, 128), lambda i: (i, 0)),
            pl.BlockSpec((8, 128), lambda i: (0, 0)),
        ],
        out_shape=[
            jax.ShapeDtypeStruct((N, 128), jnp.float32),
            jax.ShapeDtypeStruct((8, 128), jnp.float32),
        ],
    )(outp, den, xr, wbo, wbx)


def _fin2_body(out2_ref, psum_ref, g_ref, b_ref, y_ref):
    mean = psum_ref[0:1, :] / float(N)
    var = psum_ref[1:2, :] / float(N) - mean * mean
    rstd = lax.rsqrt(var + 1e-5)
    y = (out2_ref[...] - mean) * rstd * g_ref[...] + b_ref[...]
    y_ref[...] = jnp.where(y >= 0.0, y, 0.01 * y)


def _fin2(out2, psum, gamma, beta):
    BN = 1000
    return pl.pallas_call(
        _fin2_body,
        grid=(N // BN,),
        in_specs=[
            pl.BlockSpec((BN, 128), lambda i: (i, 0)),
            pl.BlockSpec((8, 128), lambda i: (0, 0)),
            pl.BlockSpec((1, 128), lambda i: (0, 0)),
            pl.BlockSpec((1, 128), lambda i: (0, 0)),
        ],
        out_specs=pl.BlockSpec((BN, 128), lambda i: (i, 0)),
        out_shape=jax.ShapeDtypeStruct((N, 128), jnp.float32),
    )(out2, psum, gamma, beta)


def kernel(x, edge_index, edge_attr, Wq, bq, Wk, bk, Wv, bv, We, Wskip, bskip,
           Wbeta, bn_gamma, bn_beta):
    Wcat = jnp.concatenate([Wq * 0.25, Wk, Wv, Wskip], axis=1)
    bcat = jnp.concatenate([bq * 0.25, bk, bv, bskip])[None, :]
    wbo = (Wbeta[0:128, 0] + Wbeta[256:384, 0])[None, :]
    wbx = (Wbeta[128:256, 0] - Wbeta[256:384, 0])[None, :]

    src_r = edge_index[0].reshape(NW, CHUNKS, B)
    dst_r = edge_index[1].reshape(NW, CHUNKS, B)
    dst8_r = jnp.right_shift(dst_r, 3)
    doff_r = jnp.left_shift(jnp.bitwise_and(dst_r, 7), 4)
    idx_pack = jnp.stack([src_r, dst_r, dst8_r, doff_r], axis=2)
    zer128 = jnp.zeros((NP, 128), jnp.float32)

    q4, ktab, vtab, xr = _proj(x, Wcat, bcat)
    etab = _eproj(edge_attr, We)

    den128, outp = _sc_edges(q4, ktab, vtab, etab, idx_pack, zer128)
    denoms = den128.reshape(NC, NP, 16)

    out2, psum = _fin1(outp, denoms, xr, wbo, wbx)
    return _fin2(out2, psum, bn_gamma[None, :], bn_beta[None, :])


# speedup vs baseline: 1.0262x; 1.0262x over previous
import functools

import jax
import jax.numpy as jnp
from jax import lax
from jax.experimental import pallas as pl
from jax.experimental.pallas import tpu as pltpu
from jax.experimental.pallas import tpu_sc as plsc

N = 10000
E = 320000
D_MODEL = 128
H = 8
DH = 16
D_EDGE = 16

NC = 2
NS = 16
NW = NC * NS
B = 40
CHUNKS = E // (NW * B)
EPW = E // NW
NP = 10240
RPS = NP // NS


def _ds16(h):
    return pl.ds(h * 16, 16)


def _proj_body(x_ref, w_ref, b_ref, q_ref, k_ref, v_ref, xr_ref):
    p = jnp.dot(x_ref[...], w_ref[...], preferred_element_type=jnp.float32)
    p = p + b_ref[...]
    q_ref[...] = p[:, 0:128]
    k_ref[...] = p[:, 128:256]
    v_ref[...] = p[:, 256:384]
    xr_ref[...] = p[:, 384:512]


def _proj(x, Wcat, bcat):
    BN = 1000
    outs = [jax.ShapeDtypeStruct((N, 128), jnp.float32)] * 4
    return pl.pallas_call(
        _proj_body,
        grid=(N // BN,),
        in_specs=[
            pl.BlockSpec((BN, 128), lambda i: (i, 0)),
            pl.BlockSpec((128, 512), lambda i: (0, 0)),
            pl.BlockSpec((1, 512), lambda i: (0, 0)),
        ],
        out_specs=[pl.BlockSpec((BN, 128), lambda i: (i, 0))] * 4,
        out_shape=outs,
    )(x, Wcat, bcat)


def _eproj_body(a_ref, w_ref, o_ref):
    o_ref[...] = jnp.dot(a_ref[...], w_ref[...],
                         preferred_element_type=jnp.float32)


def _eproj(edge_attr, We):
    BE = 3200
    return pl.pallas_call(
        _eproj_body,
        grid=(E // BE,),
        in_specs=[
            pl.BlockSpec((BE, 16), lambda i: (i, 0)),
            pl.BlockSpec((16, 128), lambda i: (0, 0)),
        ],
        out_specs=pl.BlockSpec((BE, 128), lambda i: (i, 0)),
        out_shape=jax.ShapeDtypeStruct((E, 128), jnp.float32),
    )(edge_attr, We)


def _sc_body(q4_hbm, k_hbm, v_hbm, e_hbm, idx_hbm, zer128_hbm,
             den_out, outp_hbm,
             den_sh, acc_sh, buf1, buf2, buf3, vbuf, exbuf, denbuf, idxb,
             sem1, sem2, semA, semD):
    c = lax.axis_index("c")
    s = lax.axis_index("s")
    wid = c * NS + s
    rows = pl.ds(s * RPS, RPS)
    drows = pl.ds(s * (NP // 8 // NS), NP // 8 // NS)

    pltpu.sync_copy(zer128_hbm.at[drows], den_sh.at[drows])
    pltpu.sync_copy(zer128_hbm.at[rows], acc_sh.at[rows])

    lane = lax.iota(jnp.int32, 16)
    zv = jnp.zeros((16,), jnp.float32)

    plsc.subcore_barrier()

    def _chunk(t, _):
        base = wid * EPW + t * B

        @pl.when(t > 0)
        def _():
            pltpu.make_async_copy(buf1, acc_sh.at[idxb.at[1]], semA).wait()
            pltpu.make_async_copy(denbuf, den_sh.at[idxb.at[2]], semD).wait()

        pltpu.sync_copy(idx_hbm.at[wid, t], idxb)
        cq = pltpu.async_copy(q4_hbm.at[idxb.at[1]], buf1, sem1)
        ck = pltpu.async_copy(k_hbm.at[idxb.at[0]], buf3, sem1)
        ce = pltpu.async_copy(e_hbm.at[pl.ds(base, B)], buf2, sem1)
        cv = pltpu.async_copy(v_hbm.at[idxb.at[0]], vbuf, sem2)
        cq.wait()
        ck.wait()
        ce.wait()

        def _edge(i, _):
            av = jnp.zeros((16,), jnp.float32)
            for h in range(H):
                qv = buf1[i, _ds16(h)]
                w = buf2[i, _ds16(h)] + buf3[i, _ds16(h)]
                av = jnp.where(lane == h, jnp.sum(qv * w), av)
            exbuf[i, :] = jnp.exp(av)
            return _
        lax.fori_loop(0, B, _edge, None)

        for j in (0, 16, B - 16):
            ov = idxb[3, pl.ds(j, 16)]
            for r in range(16):
                i = j + r
                ev = exbuf[i, :]
                for blk in range(8):
                    denbuf[i, _ds16(blk)] = zv
                denbuf[i, pl.ds(ov[r], 16)] = ev

        pltpu.async_copy(denbuf, den_sh.at[idxb.at[2]], semD, add=True)
        cv.wait()

        def _scale(i, _):
            tv = exbuf[i, :]
            for h in range(H):
                buf1[i, _ds16(h)] = tv[h] * (vbuf[i, _ds16(h)]
                                             + buf2[i, _ds16(h)])
            return _
        lax.fori_loop(0, B, _scale, None)

        pltpu.async_copy(buf1, acc_sh.at[idxb.at[1]], semA, add=True)
        return _
    lax.fori_loop(0, CHUNKS, _chunk, None)

    pltpu.make_async_copy(buf1, acc_sh.at[idxb.at[1]], semA).wait()
    pltpu.make_async_copy(denbuf, den_sh.at[idxb.at[2]], semD).wait()

    plsc.subcore_barrier()
    pltpu.sync_copy(den_sh.at[drows], den_out.at[c, drows])
    pltpu.sync_copy(acc_sh.at[rows], outp_hbm.at[c, rows])


def _sc_edges(q4, ktab, vtab, etab, idx_pack, zer128):
    mesh = plsc.VectorSubcoreMesh(core_axis_name="c", subcore_axis_name="s")
    f = pl.kernel(
        _sc_body,
        out_type=(jax.ShapeDtypeStruct((NC, NP // 8, 128), jnp.float32),
                  jax.ShapeDtypeStruct((NC, NP, 128), jnp.float32)),
        mesh=mesh,
        compiler_params=pltpu.CompilerParams(needs_layout_passes=False),
        scratch_types=[
            pltpu.VMEM_SHARED((NP // 8, 128), jnp.float32),
            pltpu.VMEM_SHARED((NP, 128), jnp.float32),
            pltpu.VMEM((B, 128), jnp.float32),
            pltpu.VMEM((B, 128), jnp.float32),
            pltpu.VMEM((B, 128), jnp.float32),
            pltpu.VMEM((B, 128), jnp.float32),
            pltpu.VMEM((B, 16), jnp.float32),
            pltpu.VMEM((B, 128), jnp.float32),
            pltpu.VMEM((4, B), jnp.int32),
            pltpu.SemaphoreType.DMA,
            pltpu.SemaphoreType.DMA,
            pltpu.SemaphoreType.DMA,
            pltpu.SemaphoreType.DMA,
        ],
    )
    return f(q4, ktab, vtab, etab, idx_pack, zer128)


def _fin1_body(outp_ref, den_ref, xr_ref, wbo_ref, wbx_ref, out2_ref,
               psum_ref):
    i = pl.program_id(0)
    dsum = den_ref[0] + den_ref[1] + 1e-16
    dexp = jnp.repeat(dsum[:, 0:8], 16, axis=1)
    o = (outp_ref[0] + outp_ref[1]) / dexp
    xr = xr_ref[...]
    t = jnp.sum(o * wbo_ref[...] + xr * wbx_ref[...], axis=1, keepdims=True)
    g = 1.0 / (1.0 + jnp.exp(-t))
    out2 = g * xr + (1.0 - g) * o
    out2_ref[...] = out2

    @pl.when(i == 0)
    def _():
        psum_ref[...] = jnp.zeros((8, 128), jnp.float32)

    psum_ref[0:1, :] += jnp.sum(out2, axis=0, keepdims=True)
    psum_ref[1:2, :] += jnp.sum(out2 * out2, axis=0, keepdims=True)


def _fin1(outp, den, xr, wbo, wbx):
    BN = 2000
    return pl.pallas_call(
        _fin1_body,
        grid=(N // BN,),
        in_specs=[
            pl.BlockSpec((2, BN, 128), lambda i: (0, i, 0)),
            pl.BlockSpec((2, BN, 16), lambda i: (0, i, 0)),
            pl.BlockSpec((BN, 128), lambda i: (i, 0)),
            pl.BlockSpec((1, 128), lambda i: (0, 0)),
            pl.BlockSpec((1, 128), lambda i: (0, 0)),
        ],
        out_specs=[
            pl.BlockSpec((BN, 128), lambda i: (i, 0)),
            pl.BlockSpec((8, 128), lambda i: (0, 0)),
        ],
        out_shape=[
            jax.ShapeDtypeStruct((N, 128), jnp.float32),
            jax.ShapeDtypeStruct((8, 128), jnp.float32),
        ],
    )(outp, den, xr, wbo, wbx)


def _fin2_body(out2_ref, psum_ref, g_ref, b_ref, y_ref):
    mean = psum_ref[0:1, :] / float(N)
    var = psum_ref[1:2, :] / float(N) - mean * mean
    rstd = lax.rsqrt(var + 1e-5)
    y = (out2_ref[...] - mean) * rstd * g_ref[...] + b_ref[...]
    y_ref[...] = jnp.where(y >= 0.0, y, 0.01 * y)


def _fin2(out2, psum, gamma, beta):
    BN = 1000
    return pl.pallas_call(
        _fin2_body,
        grid=(N // BN,),
        in_specs=[
            pl.BlockSpec((BN, 128), lambda i: (i, 0)),
            pl.BlockSpec((8, 128), lambda i: (0, 0)),
            pl.BlockSpec((1, 128), lambda i: (0, 0)),
            pl.BlockSpec((1, 128), lambda i: (0, 0)),
        ],
        out_specs=pl.BlockSpec((BN, 128), lambda i: (i, 0)),
        out_shape=jax.ShapeDtypeStruct((N, 128), jnp.float32),
    )(out2, psum, gamma, beta)


def kernel(x, edge_index, edge_attr, Wq, bq, Wk, bk, Wv, bv, We, Wskip, bskip,
           Wbeta, bn_gamma, bn_beta):
    Wcat = jnp.concatenate([Wq * 0.25, Wk, Wv, Wskip], axis=1)
    bcat = jnp.concatenate([bq * 0.25, bk, bv, bskip])[None, :]
    wbo = (Wbeta[0:128, 0] + Wbeta[256:384, 0])[None, :]
    wbx = (Wbeta[128:256, 0] - Wbeta[256:384, 0])[None, :]

    src_r = edge_index[0].reshape(NW, CHUNKS, B)
    dst_r = edge_index[1].reshape(NW, CHUNKS, B)
    dst8_r = jnp.right_shift(dst_r, 3)
    doff_r = jnp.left_shift(jnp.bitwise_and(dst_r, 7), 4)
    idx_pack = jnp.stack([src_r, dst_r, dst8_r, doff_r], axis=2)
    zer128 = jnp.zeros((NP, 128), jnp.float32)

    q4, ktab, vtab, xr = _proj(x, Wcat, bcat)
    etab = _eproj(edge_attr, We)

    den128, outp = _sc_edges(q4, ktab, vtab, etab, idx_pack, zer128)
    denoms = den128.reshape(NC, NP, 16)

    out2, psum = _fin1(outp, denoms, xr, wbo, wbx)
    return _fin2(out2, psum, bn_gamma[None, :], bn_beta[None, :])


# speedup vs baseline: 1.1224x; 1.0938x over previous
import functools

import jax
import jax.numpy as jnp
from jax import lax
from jax.experimental import pallas as pl
from jax.experimental.pallas import tpu as pltpu
from jax.experimental.pallas import tpu_sc as plsc

N = 10000
E = 320000
D_MODEL = 128
H = 8
DH = 16
D_EDGE = 16

NC = 2
NS = 16
NW = NC * NS
B = 40
CHUNKS = E // (NW * B)
EPW = E // NW
NP = 10240
RPS = NP // NS


def _ds16(h):
    return pl.ds(h * 16, 16)


def _proj_body(x_ref, w_ref, b_ref, q_ref, k_ref, v_ref, xr_ref):
    p = jnp.dot(x_ref[...], w_ref[...], preferred_element_type=jnp.float32)
    p = p + b_ref[...]
    q_ref[...] = p[:, 0:128]
    k_ref[...] = p[:, 128:256]
    v_ref[...] = p[:, 256:384]
    xr_ref[...] = p[:, 384:512]


def _proj(x, Wcat, bcat):
    BN = 1000
    outs = [jax.ShapeDtypeStruct((N, 128), jnp.float32)] * 4
    return pl.pallas_call(
        _proj_body,
        grid=(N // BN,),
        in_specs=[
            pl.BlockSpec((BN, 128), lambda i: (i, 0)),
            pl.BlockSpec((128, 512), lambda i: (0, 0)),
            pl.BlockSpec((1, 512), lambda i: (0, 0)),
        ],
        out_specs=[pl.BlockSpec((BN, 128), lambda i: (i, 0))] * 4,
        out_shape=outs,
    )(x, Wcat, bcat)


def _eproj_body(a_ref, w_ref, o_ref):
    o_ref[...] = jnp.dot(a_ref[...], w_ref[...],
                         preferred_element_type=jnp.float32)


def _eproj(edge_attr, We):
    BE = 3200
    return pl.pallas_call(
        _eproj_body,
        grid=(E // BE,),
        in_specs=[
            pl.BlockSpec((BE, 16), lambda i: (i, 0)),
            pl.BlockSpec((16, 128), lambda i: (0, 0)),
        ],
        out_specs=pl.BlockSpec((BE, 128), lambda i: (i, 0)),
        out_shape=jax.ShapeDtypeStruct((E, 128), jnp.float32),
    )(edge_attr, We)


def _sc_body(q4_hbm, k_hbm, v_hbm, e_hbm, idx_hbm, zer128_hbm,
             den_out, outp_hbm,
             den_sh, acc_sh, buf1, buf2, buf3, vbuf, exbuf, denbuf, idxbA,
             idxbB, sem1, sem2, semA, semD):
    c = lax.axis_index("c")
    s = lax.axis_index("s")
    wid = c * NS + s
    rows = pl.ds(s * RPS, RPS)
    drows = pl.ds(s * (NP // 8 // NS), NP // 8 // NS)

    pltpu.sync_copy(zer128_hbm.at[drows], den_sh.at[drows])
    pltpu.sync_copy(zer128_hbm.at[rows], acc_sh.at[rows])

    lane = lax.iota(jnp.int32, 16)
    zv = jnp.zeros((16,), jnp.float32)

    plsc.subcore_barrier()

    def _body(t, idxb, first):
        base = wid * EPW + t * B

        @pl.when(jnp.logical_not(first))
        def _():
            pltpu.make_async_copy(vbuf, acc_sh.at[idxb.at[1]], semA).wait()
            pltpu.make_async_copy(denbuf, den_sh.at[idxb.at[2]], semD).wait()

        cq = pltpu.async_copy(q4_hbm.at[idxb.at[1]], buf1, sem1)
        ck = pltpu.async_copy(k_hbm.at[idxb.at[0]], buf3, sem1)
        ce = pltpu.async_copy(e_hbm.at[pl.ds(base, B)], buf2, sem1)
        cv = pltpu.async_copy(v_hbm.at[idxb.at[0]], vbuf, sem2)
        cq.wait()
        ck.wait()
        ce.wait()
        cv.wait()

        def _edge(i, _):
            av = jnp.zeros((16,), jnp.float32)
            for h in range(H):
                qv = buf1[i, _ds16(h)]
                w = buf2[i, _ds16(h)] + buf3[i, _ds16(h)]
                av = jnp.where(lane == h, jnp.sum(qv * w), av)
            ev = jnp.exp(av)
            exbuf[i, :] = ev
            for h in range(H):
                vbuf[i, _ds16(h)] = ev[h] * (vbuf[i, _ds16(h)]
                                             + buf2[i, _ds16(h)])
            return _
        lax.fori_loop(0, B, _edge, None)

        for j in (0, 16, B - 16):
            ov = idxb[3, pl.ds(j, 16)]
            for r in range(16):
                i = j + r
                ev = exbuf[i, :]
                for blk in range(8):
                    denbuf[i, _ds16(blk)] = zv
                denbuf[i, pl.ds(ov[r], 16)] = ev

        pltpu.async_copy(denbuf, den_sh.at[idxb.at[2]], semD, add=True)
        pltpu.async_copy(vbuf, acc_sh.at[idxb.at[1]], semA, add=True)

    pltpu.sync_copy(idx_hbm.at[wid, 0], idxbA)
    last = CHUNKS - 1

    def _pair(t2, _):
        t = 2 * t2
        ci1 = pltpu.async_copy(idx_hbm.at[wid, jnp.minimum(t + 1, last)],
                               idxbB, sem1)
        _body(t, idxbA, t2 == 0)
        ci1.wait()
        ci2 = pltpu.async_copy(idx_hbm.at[wid, jnp.minimum(t + 2, last)],
                               idxbA, sem1)
        _body(t + 1, idxbB, False)
        ci2.wait()
        return _
    lax.fori_loop(0, CHUNKS // 2, _pair, None)

    pltpu.make_async_copy(vbuf, acc_sh.at[idxbB.at[1]], semA).wait()
    pltpu.make_async_copy(denbuf, den_sh.at[idxbB.at[2]], semD).wait()

    plsc.subcore_barrier()
    pltpu.sync_copy(den_sh.at[drows], den_out.at[c, drows])
    pltpu.sync_copy(acc_sh.at[rows], outp_hbm.at[c, rows])


def _sc_edges(q4, ktab, vtab, etab, idx_pack, zer128):
    mesh = plsc.VectorSubcoreMesh(core_axis_name="c", subcore_axis_name="s")
    f = pl.kernel(
        _sc_body,
        out_type=(jax.ShapeDtypeStruct((NC, NP // 8, 128), jnp.float32),
                  jax.ShapeDtypeStruct((NC, NP, 128), jnp.float32)),
        mesh=mesh,
        compiler_params=pltpu.CompilerParams(needs_layout_passes=False),
        scratch_types=[
            pltpu.VMEM_SHARED((NP // 8, 128), jnp.float32),
            pltpu.VMEM_SHARED((NP, 128), jnp.float32),
            pltpu.VMEM((B, 128), jnp.float32),
            pltpu.VMEM((B, 128), jnp.float32),
            pltpu.VMEM((B, 128), jnp.float32),
            pltpu.VMEM((B, 128), jnp.float32),
            pltpu.VMEM((B, 16), jnp.float32),
            pltpu.VMEM((B, 128), jnp.float32),
            pltpu.VMEM((4, B), jnp.int32),
            pltpu.VMEM((4, B), jnp.int32),
            pltpu.SemaphoreType.DMA,
            pltpu.SemaphoreType.DMA,
            pltpu.SemaphoreType.DMA,
            pltpu.SemaphoreType.DMA,
        ],
    )
    return f(q4, ktab, vtab, etab, idx_pack, zer128)


def _fin1_body(outp_ref, den_ref, xr_ref, wbo_ref, wbx_ref, out2_ref,
               psum_ref):
    i = pl.program_id(0)
    dsum = den_ref[0] + den_ref[1] + 1e-16
    dexp = jnp.repeat(dsum[:, 0:8], 16, axis=1)
    o = (outp_ref[0] + outp_ref[1]) / dexp
    xr = xr_ref[...]
    t = jnp.sum(o * wbo_ref[...] + xr * wbx_ref[...], axis=1, keepdims=True)
    g = 1.0 / (1.0 + jnp.exp(-t))
    out2 = g * xr + (1.0 - g) * o
    out2_ref[...] = out2

    @pl.when(i == 0)
    def _():
        psum_ref[...] = jnp.zeros((8, 128), jnp.float32)

    psum_ref[0:1, :] += jnp.sum(out2, axis=0, keepdims=True)
    psum_ref[1:2, :] += jnp.sum(out2 * out2, axis=0, keepdims=True)


def _fin1(outp, den, xr, wbo, wbx):
    BN = 2000
    return pl.pallas_call(
        _fin1_body,
        grid=(N // BN,),
        in_specs=[
            pl.BlockSpec((2, BN, 128), lambda i: (0, i, 0)),
            pl.BlockSpec((2, BN, 16), lambda i: (0, i, 0)),
            pl.BlockSpec((BN, 128), lambda i: (i, 0)),
            pl.BlockSpec((1, 128), lambda i: (0, 0)),
            pl.BlockSpec((1, 128), lambda i: (0, 0)),
        ],
        out_specs=[
            pl.BlockSpec((BN, 128), lambda i: (i, 0)),
            pl.BlockSpec((8, 128), lambda i: (0, 0)),
        ],
        out_shape=[
            jax.ShapeDtypeStruct((N, 128), jnp.float32),
            jax.ShapeDtypeStruct((8, 128), jnp.float32),
        ],
    )(outp, den, xr, wbo, wbx)


def _fin2_body(out2_ref, psum_ref, g_ref, b_ref, y_ref):
    mean = psum_ref[0:1, :] / float(N)
    var = psum_ref[1:2, :] / float(N) - mean * mean
    rstd = lax.rsqrt(var + 1e-5)
    y = (out2_ref[...] - mean) * rstd * g_ref[...] + b_ref[...]
    y_ref[...] = jnp.where(y >= 0.0, y, 0.01 * y)


def _fin2(out2, psum, gamma, beta):
    BN = 1000
    return pl.pallas_call(
        _fin2_body,
        grid=(N // BN,),
        in_specs=[
            pl.BlockSpec((BN, 128), lambda i: (i, 0)),
            pl.BlockSpec((8, 128), lambda i: (0, 0)),
            pl.BlockSpec((1, 128), lambda i: (0, 0)),
            pl.BlockSpec((1, 128), lambda i: (0, 0)),
        ],
        out_specs=pl.BlockSpec((BN, 128), lambda i: (i, 0)),
        out_shape=jax.ShapeDtypeStruct((N, 128), jnp.float32),
    )(out2, psum, gamma, beta)


def kernel(x, edge_index, edge_attr, Wq, bq, Wk, bk, Wv, bv, We, Wskip, bskip,
           Wbeta, bn_gamma, bn_beta):
    Wcat = jnp.concatenate([Wq * 0.25, Wk, Wv, Wskip], axis=1)
    bcat = jnp.concatenate([bq * 0.25, bk, bv, bskip])[None, :]
    wbo = (Wbeta[0:128, 0] + Wbeta[256:384, 0])[None, :]
    wbx = (Wbeta[128:256, 0] - Wbeta[256:384, 0])[None, :]

    src_r = edge_index[0].reshape(NW, CHUNKS, B)
    dst_r = edge_index[1].reshape(NW, CHUNKS, B)
    dst8_r = jnp.right_shift(dst_r, 3)
    doff_r = jnp.left_shift(jnp.bitwise_and(dst_r, 7), 4)
    idx_pack = jnp.stack([src_r, dst_r, dst8_r, doff_r], axis=2)
    zer128 = jnp.zeros((NP, 128), jnp.float32)

    q4, ktab, vtab, xr = _proj(x, Wcat, bcat)
    etab = _eproj(edge_attr, We)

    den128, outp = _sc_edges(q4, ktab, vtab, etab, idx_pack, zer128)
    denoms = den128.reshape(NC, NP, 16)

    out2, psum = _fin1(outp, denoms, xr, wbo, wbx)
    return _fin2(out2, psum, bn_gamma[None, :], bn_beta[None, :])
